# baseline (speedup 1.0000x reference)
import functools

import jax
import jax.numpy as jnp
from jax import lax
from jax.experimental import pallas as pl
from jax.experimental.pallas import tpu as pltpu
from jax.experimental.pallas import tpu_sc as plsc

N = 10000
E = 320000
D_IN = 128
HID = 256
G = 128

NC = 2
NS = 16
CHUNK = 128

E_PAD = ((E + NC * NS * CHUNK * 2 - 1) // (NC * NS * CHUNK * 2)) * (
    NC * NS * CHUNK * 2
)
ROWS_PAD = ((N + 1 + NS * 8 - 1) // (NS * 8)) * (NS * 8)
RPT = ROWS_PAD // NS

_EPT_DEG = E_PAD // (NC * NS)
_NCHUNK_DEG = _EPT_DEG // CHUNK


def _deg_body(
    dst2_hbm, ones_hbm, z128_hbm, out_hbm, didx_v, ones_v, acc_sh, isd0, isd1
):
    c = lax.axis_index("c")
    s = lax.axis_index("s")
    pltpu.sync_copy(ones_hbm, ones_v)
    pltpu.sync_copy(
        z128_hbm.at[pl.ds(s * RPT, RPT)], acc_sh.at[pl.ds(s * RPT, RPT)]
    )
    plsc.subcore_barrier()
    isd = (isd0, isd1)
    base_r = (c * NS + s) * _NCHUNK_DEG

    pltpu.sync_copy(dst2_hbm.at[base_r], didx_v.at[0])
    pltpu.async_copy(dst2_hbm.at[base_r + 1], didx_v.at[1], isd[1])

    def group(g, carry):
        for b in (0, 1):
            nb = 1 - b
            k = 2 * g + b

            def wait_next():
                pltpu.make_async_copy(
                    dst2_hbm.at[base_r + k + 1], didx_v.at[nb], isd[nb]
                ).wait()

            if b == 0:
                wait_next()
            else:
                pl.when(g < _NCHUNK_DEG // 2 - 1)(wait_next)

            pltpu.sync_copy(ones_v, acc_sh.at[didx_v.at[b]], add=True)

            @pl.when(g < _NCHUNK_DEG // 2 - 1)
            def _():
                pltpu.async_copy(
                    dst2_hbm.at[base_r + k + 2], didx_v.at[b], isd[b]
                )

        return carry

    lax.fori_loop(0, _NCHUNK_DEG // 2, group, 0)
    plsc.subcore_barrier()
    pltpu.sync_copy(
        acc_sh.at[pl.ds(s * RPT, RPT)],
        out_hbm.at[pl.ds(c * ROWS_PAD + s * RPT, RPT)],
    )


_EPT_SC = E_PAD // NS
_NCHUNK_SC = _EPT_SC // CHUNK


_EROWS = E_PAD // CHUNK
_NCHT = _EROWS // NS


def _scatter_body(
    m_hbm, src2_hbm, dst2_hbm, z128_hbm, out_hbm,
    sidx_v, didx_v, rows_v, acc_sh,
    iss0, isd0, iss1, isd1, gs0, gs1,
):
    c = lax.axis_index("c")
    s = lax.axis_index("s")
    pltpu.sync_copy(
        z128_hbm.at[pl.ds(s * RPT, RPT)], acc_sh.at[pl.ds(s * RPT, RPT)]
    )
    plsc.subcore_barrier()
    iss = (iss0, iss1)
    isd = (isd0, isd1)
    gs = (gs0, gs1)
    r0 = s * _NCHT
    sb = c * _EROWS + r0

    pltpu.sync_copy(src2_hbm.at[sb], sidx_v.at[0])
    pltpu.sync_copy(dst2_hbm.at[r0], didx_v.at[0])
    pltpu.async_copy(m_hbm.at[sidx_v.at[0]], rows_v.at[0], gs[0])
    pltpu.async_copy(src2_hbm.at[sb + 1], sidx_v.at[1], iss[1])
    pltpu.async_copy(dst2_hbm.at[r0 + 1], didx_v.at[1], isd[1])

    def group(g, carry):
        for b in (0, 1):
            nb = 1 - b
            k = 2 * g + b

            def stage_next():
                pltpu.make_async_copy(
                    src2_hbm.at[sb + k + 1], sidx_v.at[nb], iss[nb]
                ).wait()
                pltpu.make_async_copy(
                    dst2_hbm.at[r0 + k + 1], didx_v.at[nb], isd[nb]
                ).wait()
                pltpu.async_copy(m_hbm.at[sidx_v.at[nb]], rows_v.at[nb], gs[nb])

            if b == 0:
                stage_next()
            else:
                pl.when(g < _NCHT // 2 - 1)(stage_next)

            pltpu.make_async_copy(
                m_hbm.at[sidx_v.at[b]], rows_v.at[b], gs[b]
            ).wait()
            pltpu.sync_copy(rows_v.at[b], acc_sh.at[didx_v.at[b]], add=True)

            @pl.when(g < _NCHT // 2 - 1)
            def _():
                pltpu.async_copy(src2_hbm.at[sb + k + 2], sidx_v.at[b], iss[b])
                pltpu.async_copy(dst2_hbm.at[r0 + k + 2], didx_v.at[b], isd[b])

        return carry

    lax.fori_loop(0, _NCHT // 2, group, 0)
    plsc.subcore_barrier()
    pltpu.sync_copy(
        acc_sh.at[pl.ds(s * RPT, RPT)],
        out_hbm.at[pl.ds(c * ROWS_PAD + s * RPT, RPT)],
    )


@functools.cache
def _sc_kernels():
    mesh = plsc.VectorSubcoreMesh(
        core_axis_name="c", subcore_axis_name="s", num_cores=NC, num_subcores=NS
    )
    deg_k = pl.kernel(
        _deg_body,
        out_type=jax.ShapeDtypeStruct((NC * ROWS_PAD, 128), jnp.float32),
        mesh=mesh,
        scratch_types=[
            pltpu.VMEM((2, CHUNK), jnp.int32),
            pltpu.VMEM((CHUNK, 128), jnp.float32),
            pltpu.VMEM_SHARED((ROWS_PAD, 128), jnp.float32),
        ] + [pltpu.SemaphoreType.DMA] * 2,
    )
    scat_k = pl.kernel(
        _scatter_body,
        out_type=jax.ShapeDtypeStruct((NC * ROWS_PAD, 128), jnp.float32),
        mesh=mesh,
        scratch_types=[
            pltpu.VMEM((2, CHUNK), jnp.int32),
            pltpu.VMEM((2, CHUNK), jnp.int32),
            pltpu.VMEM((2, CHUNK, 128), jnp.float32),
            pltpu.VMEM_SHARED((ROWS_PAD, 128), jnp.float32),
        ] + [pltpu.SemaphoreType.DMA] * 6,
    )
    return deg_k, scat_k


BN = 2000
NG = N // BN


def _dinv_from_degp(degp_blk):
    deg = degp_blk[0, :, 0:1] + degp_blk[1, :, 0:1] + 1.0
    return lax.rsqrt(jnp.maximum(deg, 1.0))


def _l1_body(x_ref, w_ref, degp_ref, m_ref):
    dinv = _dinv_from_degp(degp_ref[...])
    h = jnp.dot(x_ref[...], w_ref[...], preferred_element_type=jnp.float32)
    m = h * dinv
    m_ref[0] = m[:, :128]
    m_ref[1] = m[:, 128:]


_l1_call = pl.pallas_call(
    _l1_body,
    grid=(NG,),
    in_specs=[
        pl.BlockSpec((BN, D_IN), lambda i: (i, 0)),
        pl.BlockSpec((D_IN, HID), lambda i: (0, 0)),
        pl.BlockSpec((NC, BN, 128), lambda i: (0, i, 0)),
    ],
    out_specs=pl.BlockSpec((NC, BN, 128), lambda i: (0, i, 0)),
    out_shape=jax.ShapeDtypeStruct((NC, N, 128), jnp.float32),
)


def _mid_body(S_ref, m_ref, degp_ref, b_ref, w_ref, out_ref):
    dinv = _dinv_from_degp(degp_ref[...])
    b = b_ref[...]
    lo = (S_ref[0] + m_ref[0]) * dinv + b[:, :128]
    hi = (S_ref[1] + m_ref[1]) * dinv + b[:, 128:]
    h1 = jax.nn.relu(jnp.concatenate([lo, hi], axis=1))
    m2 = jnp.dot(h1, w_ref[...], preferred_element_type=jnp.float32) * dinv
    out_ref[0] = m2[:, :128]
    out_ref[1] = m2[:, 128:]


_mid_call = pl.pallas_call(
    _mid_body,
    grid=(NG,),
    in_specs=[
        pl.BlockSpec((NC, BN, 128), lambda i: (0, i, 0)),
        pl.BlockSpec((NC, BN, 128), lambda i: (0, i, 0)),
        pl.BlockSpec((NC, BN, 128), lambda i: (0, i, 0)),
        pl.BlockSpec((1, HID), lambda i: (0, 0)),
        pl.BlockSpec((HID, HID), lambda i: (0, 0)),
    ],
    out_specs=pl.BlockSpec((NC, BN, 128), lambda i: (0, i, 0)),
    out_shape=jax.ShapeDtypeStruct((NC, N, 128), jnp.float32),
)


def _fin_body(S_ref, m_ref, degp_ref, b_ref, batch_ref, pool_ref, cnt_ref):
    i = pl.program_id(0)
    dinv = _dinv_from_degp(degp_ref[...])
    b = b_ref[...]
    lo = (S_ref[0] + m_ref[0]) * dinv + b[:, :128]
    hi = (S_ref[1] + m_ref[1]) * dinv + b[:, 128:]
    h2 = jax.nn.relu(jnp.concatenate([lo, hi], axis=1))
    bt = batch_ref[...]
    gids = lax.broadcasted_iota(jnp.int32, (1, G), 1)
    P = (bt == gids).astype(jnp.float32)
    dn = (((0,), (0,)), ((), ()))
    psum = lax.dot_general(P, h2, dn, preferred_element_type=jnp.float32)
    ones_col = jnp.ones((BN, 1), jnp.float32)
    pcnt = lax.dot_general(P, ones_col, dn, preferred_element_type=jnp.float32)

    @pl.when(i == 0)
    def _():
        pool_ref[...] = psum
        cnt_ref[...] = pcnt

    @pl.when(i > 0)
    def _():
        pool_ref[...] += psum
        cnt_ref[...] += pcnt

    @pl.when(i == NG - 1)
    def _():
        pool_ref[...] = pool_ref[...] / jnp.maximum(cnt_ref[...], 1.0)


_fin_call = pl.pallas_call(
    _fin_body,
    grid=(NG,),
    in_specs=[
        pl.BlockSpec((NC, BN, 128), lambda i: (0, i, 0)),
        pl.BlockSpec((NC, BN, 128), lambda i: (0, i, 0)),
        pl.BlockSpec((NC, BN, 128), lambda i: (0, i, 0)),
        pl.BlockSpec((1, HID), lambda i: (0, 0)),
        pl.BlockSpec((BN, 1), lambda i: (i, 0)),
    ],
    out_specs=[
        pl.BlockSpec((G, HID), lambda i: (0, 0)),
        pl.BlockSpec((G, 1), lambda i: (0, 0)),
    ],
    out_shape=[
        jax.ShapeDtypeStruct((G, HID), jnp.float32),
        jax.ShapeDtypeStruct((G, 1), jnp.float32),
    ],
)


def kernel(x, edge_index, batch, W1, b1, W2, b2):
    pad = E_PAD - E
    srcp = jnp.concatenate([edge_index[0], jnp.zeros((pad,), jnp.int32)])
    dstp = jnp.concatenate([edge_index[1], jnp.full((pad,), N, jnp.int32)])
    src2 = jnp.concatenate([srcp, srcp + N]).reshape(NC * _EROWS, CHUNK)
    dst2 = dstp.reshape(_EROWS, CHUNK)
    ones128 = jnp.ones((CHUNK, 128), jnp.float32)
    z128 = jnp.zeros((ROWS_PAD, 128), jnp.float32)

    deg_k, scat_k = _sc_kernels()
    degp = deg_k(dst2, ones128, z128).reshape(NC, ROWS_PAD, 128)

    m1 = _l1_call(x, W1, degp)
    S1 = scat_k(m1.reshape(NC * N, 128), src2, dst2, z128)
    S1 = S1.reshape(NC, ROWS_PAD, 128)

    m2 = _mid_call(S1, m1, degp, b1.reshape(1, HID), W2)
    S2 = scat_k(m2.reshape(NC * N, 128), src2, dst2, z128)
    S2 = S2.reshape(NC, ROWS_PAD, 128)

    pool, _ = _fin_call(
        S2, m2, degp, b2.reshape(1, HID), batch.reshape(N, 1).astype(jnp.int32)
    )
    return pool


# speedup vs baseline: 2.5014x; 2.5014x over previous
import functools

import jax
import jax.numpy as jnp
from jax import lax
from jax.experimental import pallas as pl
from jax.experimental.pallas import tpu as pltpu
from jax.experimental.pallas import tpu_sc as plsc

N = 10000
E = 320000
D_IN = 128
HID = 256
G = 128

NC = 2
NS = 16
CHUNK = 128

E_PAD = ((E + NC * NS * CHUNK * 2 - 1) // (NC * NS * CHUNK * 2)) * (
    NC * NS * CHUNK * 2
)
ROWS_PAD = ((N + 1 + NS * 8 - 1) // (NS * 8)) * (NS * 8)
RPT = ROWS_PAD // NS

_EPT_DEG = E_PAD // (NC * NS)
_NCHUNK_DEG = _EPT_DEG // CHUNK


def _deg_body(
    dst2_hbm, ones_hbm, z128_hbm, out_hbm, didx_v, ones_v, acc_sh, isd0, isd1
):
    c = lax.axis_index("c")
    s = lax.axis_index("s")
    pltpu.sync_copy(ones_hbm, ones_v)
    pltpu.sync_copy(
        z128_hbm.at[pl.ds(s * RPT, RPT)], acc_sh.at[pl.ds(s * RPT, RPT)]
    )
    plsc.subcore_barrier()
    isd = (isd0, isd1)
    base_r = (c * NS + s) * _NCHUNK_DEG

    pltpu.sync_copy(dst2_hbm.at[base_r], didx_v.at[0])
    pltpu.async_copy(dst2_hbm.at[base_r + 1], didx_v.at[1], isd[1])

    def group(g, carry):
        for b in (0, 1):
            nb = 1 - b
            k = 2 * g + b

            def wait_next():
                pltpu.make_async_copy(
                    dst2_hbm.at[base_r + k + 1], didx_v.at[nb], isd[nb]
                ).wait()

            if b == 0:
                wait_next()
            else:
                pl.when(g < _NCHUNK_DEG // 2 - 1)(wait_next)

            pltpu.sync_copy(ones_v, acc_sh.at[didx_v.at[b]], add=True)

            @pl.when(g < _NCHUNK_DEG // 2 - 1)
            def _():
                pltpu.async_copy(
                    dst2_hbm.at[base_r + k + 2], didx_v.at[b], isd[b]
                )

        return carry

    lax.fori_loop(0, _NCHUNK_DEG // 2, group, 0)
    plsc.subcore_barrier()
    pltpu.sync_copy(
        acc_sh.at[pl.ds(s * RPT, RPT)],
        out_hbm.at[pl.ds(c * ROWS_PAD + s * RPT, RPT)],
    )


_EPT_SC = E_PAD // NS
_NCHUNK_SC = _EPT_SC // CHUNK


_EROWS = E_PAD // CHUNK
_NCHT = _EROWS // NS


def _scatter_body(
    m_hbm, src2_hbm, dst2_hbm, z128_hbm, out_hbm,
    sidx_v, didx_v, rows_v, acc_sh,
    iss0, isd0, iss1, isd1, gs0, gs1,
):
    c = lax.axis_index("c")
    s = lax.axis_index("s")
    pltpu.sync_copy(
        z128_hbm.at[pl.ds(s * RPT, RPT)], acc_sh.at[pl.ds(s * RPT, RPT)]
    )
    plsc.subcore_barrier()
    iss = (iss0, iss1)
    isd = (isd0, isd1)
    gs = (gs0, gs1)
    r0 = s * _NCHT
    sb = c * _EROWS + r0

    pltpu.sync_copy(src2_hbm.at[sb], sidx_v.at[0])
    pltpu.sync_copy(dst2_hbm.at[r0], didx_v.at[0])
    pltpu.async_copy(m_hbm.at[sidx_v.at[0]], rows_v.at[0], gs[0])
    pltpu.async_copy(src2_hbm.at[sb + 1], sidx_v.at[1], iss[1])
    pltpu.async_copy(dst2_hbm.at[r0 + 1], didx_v.at[1], isd[1])

    def group(g, carry):
        for b in (0, 1):
            nb = 1 - b
            k = 2 * g + b

            def stage_next():
                pltpu.make_async_copy(
                    src2_hbm.at[sb + k + 1], sidx_v.at[nb], iss[nb]
                ).wait()
                pltpu.make_async_copy(
                    dst2_hbm.at[r0 + k + 1], didx_v.at[nb], isd[nb]
                ).wait()
                pltpu.async_copy(m_hbm.at[sidx_v.at[nb]], rows_v.at[nb], gs[nb])

            if b == 0:
                stage_next()
            else:
                pl.when(g < _NCHT // 2 - 1)(stage_next)

            pltpu.make_async_copy(
                m_hbm.at[sidx_v.at[b]], rows_v.at[b], gs[b]
            ).wait()
            pltpu.sync_copy(rows_v.at[b], acc_sh.at[didx_v.at[b]], add=True)

            @pl.when(g < _NCHT // 2 - 1)
            def _():
                pltpu.async_copy(src2_hbm.at[sb + k + 2], sidx_v.at[b], iss[b])
                pltpu.async_copy(dst2_hbm.at[r0 + k + 2], didx_v.at[b], isd[b])

        return carry

    lax.fori_loop(0, _NCHT // 2, group, 0)
    plsc.subcore_barrier()
    pltpu.sync_copy(
        acc_sh.at[pl.ds(s * RPT, RPT)],
        out_hbm.at[pl.ds(c * ROWS_PAD + s * RPT, RPT)],
    )


@functools.cache
def _sc_kernels():
    mesh = plsc.VectorSubcoreMesh(
        core_axis_name="c", subcore_axis_name="s", num_cores=NC, num_subcores=NS
    )
    deg_k = pl.kernel(
        _deg_body,
        out_type=jax.ShapeDtypeStruct((NC * ROWS_PAD, 128), jnp.float32),
        mesh=mesh,
        scratch_types=[
            pltpu.VMEM((2, CHUNK), jnp.int32),
            pltpu.VMEM((CHUNK, 128), jnp.float32),
            pltpu.VMEM_SHARED((ROWS_PAD, 128), jnp.float32),
        ] + [pltpu.SemaphoreType.DMA] * 2,
    )
    scat_k = pl.kernel(
        _scatter_body,
        out_type=jax.ShapeDtypeStruct((NC * ROWS_PAD, 128), jnp.float32),
        mesh=mesh,
        scratch_types=[
            pltpu.VMEM((2, CHUNK), jnp.int32),
            pltpu.VMEM((2, CHUNK), jnp.int32),
            pltpu.VMEM((2, CHUNK, 128), jnp.float32),
            pltpu.VMEM_SHARED((ROWS_PAD, 128), jnp.float32),
        ] + [pltpu.SemaphoreType.DMA] * 6,
    )
    return deg_k, scat_k


BN = 2000
NG = N // BN


def _dinv_from_degp(degp_blk):
    deg = degp_blk[0, :, 0:1] + degp_blk[1, :, 0:1] + 1.0
    return lax.rsqrt(jnp.maximum(deg, 1.0))


def _l1_body(x_ref, w_ref, degp_ref, m_ref):
    dinv = _dinv_from_degp(degp_ref[...])
    h = jnp.dot(x_ref[...], w_ref[...], preferred_element_type=jnp.float32)
    m = h * dinv
    m_ref[0] = m[:, :128]
    m_ref[1] = m[:, 128:]


_l1_call = pl.pallas_call(
    _l1_body,
    grid=(NG,),
    in_specs=[
        pl.BlockSpec((BN, D_IN), lambda i: (i, 0)),
        pl.BlockSpec((D_IN, HID), lambda i: (0, 0)),
        pl.BlockSpec((NC, BN, 128), lambda i: (0, i, 0)),
    ],
    out_specs=pl.BlockSpec((NC, BN, 128), lambda i: (0, i, 0)),
    out_shape=jax.ShapeDtypeStruct((NC, N, 128), jnp.float32),
)


def _mid_body(S_ref, m_ref, degp_ref, b_ref, w_ref, out_ref):
    dinv = _dinv_from_degp(degp_ref[...])
    b = b_ref[...]
    lo = (S_ref[0] + m_ref[0]) * dinv + b[:, :128]
    hi = (S_ref[1] + m_ref[1]) * dinv + b[:, 128:]
    h1 = jax.nn.relu(jnp.concatenate([lo, hi], axis=1))
    m2 = jnp.dot(h1, w_ref[...], preferred_element_type=jnp.float32) * dinv
    out_ref[0] = m2[:, :128]
    out_ref[1] = m2[:, 128:]


_mid_call = pl.pallas_call(
    _mid_body,
    grid=(NG,),
    in_specs=[
        pl.BlockSpec((NC, BN, 128), lambda i: (0, i, 0)),
        pl.BlockSpec((NC, BN, 128), lambda i: (0, i, 0)),
        pl.BlockSpec((NC, BN, 128), lambda i: (0, i, 0)),
        pl.BlockSpec((1, HID), lambda i: (0, 0)),
        pl.BlockSpec((HID, HID), lambda i: (0, 0)),
    ],
    out_specs=pl.BlockSpec((NC, BN, 128), lambda i: (0, i, 0)),
    out_shape=jax.ShapeDtypeStruct((NC, N, 128), jnp.float32),
)


def _fin_body(S_ref, m_ref, degp_ref, b_ref, batch_ref, pool_ref, cnt_ref):
    i = pl.program_id(0)
    dinv = _dinv_from_degp(degp_ref[...])
    b = b_ref[...]
    lo = (S_ref[0] + m_ref[0]) * dinv + b[:, :128]
    hi = (S_ref[1] + m_ref[1]) * dinv + b[:, 128:]
    h2 = jax.nn.relu(jnp.concatenate([lo, hi], axis=1))
    bt = batch_ref[...]
    gids = lax.broadcasted_iota(jnp.int32, (1, G), 1)
    P = (bt == gids).astype(jnp.float32)
    dn = (((0,), (0,)), ((), ()))
    psum = lax.dot_general(P, h2, dn, preferred_element_type=jnp.float32)
    ones_col = jnp.ones((BN, 1), jnp.float32)
    pcnt = lax.dot_general(P, ones_col, dn, preferred_element_type=jnp.float32)

    @pl.when(i == 0)
    def _():
        pool_ref[...] = psum
        cnt_ref[...] = pcnt

    @pl.when(i > 0)
    def _():
        pool_ref[...] += psum
        cnt_ref[...] += pcnt

    @pl.when(i == NG - 1)
    def _():
        pool_ref[...] = pool_ref[...] / jnp.maximum(cnt_ref[...], 1.0)


_fin_call = pl.pallas_call(
    _fin_body,
    grid=(NG,),
    in_specs=[
        pl.BlockSpec((NC, BN, 128), lambda i: (0, i, 0)),
        pl.BlockSpec((NC, BN, 128), lambda i: (0, i, 0)),
        pl.BlockSpec((NC, BN, 128), lambda i: (0, i, 0)),
        pl.BlockSpec((1, HID), lambda i: (0, 0)),
        pl.BlockSpec((BN, 1), lambda i: (i, 0)),
    ],
    out_specs=[
        pl.BlockSpec((G, HID), lambda i: (0, 0)),
        pl.BlockSpec((G, 1), lambda i: (0, 0)),
    ],
    out_shape=[
        jax.ShapeDtypeStruct((G, HID), jnp.float32),
        jax.ShapeDtypeStruct((G, 1), jnp.float32),
    ],
)


def kernel(x, edge_index, batch, W1, b1, W2, b2):
    pad = E_PAD - E
    pidx = jnp.arange(pad, dtype=jnp.int32)
    srcp = jnp.concatenate([edge_index[0], pidx % N])
    dstp = jnp.concatenate([edge_index[1], N + pidx % (ROWS_PAD - N)])
    src2 = jnp.concatenate([srcp, srcp + N]).reshape(NC * _EROWS, CHUNK)
    dst2 = dstp.reshape(_EROWS, CHUNK)
    ones128 = jnp.ones((CHUNK, 128), jnp.float32)
    z128 = jnp.zeros((ROWS_PAD, 128), jnp.float32)

    deg_k, scat_k = _sc_kernels()
    degp = deg_k(dst2, ones128, z128).reshape(NC, ROWS_PAD, 128)

    m1 = _l1_call(x, W1, degp)
    S1 = scat_k(m1.reshape(NC * N, 128), src2, dst2, z128)
    S1 = S1.reshape(NC, ROWS_PAD, 128)

    m2 = _mid_call(S1, m1, degp, b1.reshape(1, HID), W2)
    S2 = scat_k(m2.reshape(NC * N, 128), src2, dst2, z128)
    S2 = S2.reshape(NC, ROWS_PAD, 128)

    pool, _ = _fin_call(
        S2, m2, degp, b2.reshape(1, HID), batch.reshape(N, 1).astype(jnp.int32)
    )
    return pool
